# Initial kernel scaffold; baseline (speedup 1.0000x reference)
import jax
import jax.numpy as jnp
from jax.experimental import pallas as pl


def kernel(x, edge_index, W1, b1, g1, be1, W2, b2, g2, be2):
    raise NotImplementedError("write your pallas kernel here")


# speedup vs baseline: 9.6247x; 9.6247x over previous
import functools

import jax
import jax.numpy as jnp
from jax import lax
from jax.experimental import pallas as pl
from jax.experimental.pallas import tpu as pltpu
from jax.experimental.pallas import tpu_sc as plsc

N = 10000
D = 128
EPS = 1e-5

NC = 2
NS = 16
NW = NC * NS
LANES = 16
BATCH = 128

N_ACC = 10240
DUMMY = N
RPT = N_ACC // NS

BLK = 1000

_mesh = plsc.VectorSubcoreMesh(
    core_axis_name="c", subcore_axis_name="s", num_cores=NC, num_subcores=NS)


def _mask_cols(row_v, col_v, colm_v):
  for j in range(BATCH // LANES):
    sl = pl.ds(j * LANES, LANES)
    r = row_v[sl]
    c = col_v[sl]
    colm_v[sl] = jnp.where(r == c, jnp.int32(DUMMY), c)


def _deg_body(nbatch, epw, ei, zeros128, ones128, out,
              row_v, col_v, colm_v, ones_v, acc):
  cid = lax.axis_index("c")
  sid = lax.axis_index("s")
  wid = cid * NS + sid
  my_rows = pl.ds(sid * RPT, RPT)
  pltpu.sync_copy(zeros128, acc.at[my_rows])
  pltpu.sync_copy(ones128, ones_v)
  plsc.subcore_barrier()

  def body(b, carry):
    base = pl.multiple_of(wid * epw + b * BATCH, BATCH)
    pltpu.sync_copy(ei.at[0, pl.ds(base, BATCH)], row_v)
    pltpu.sync_copy(ei.at[1, pl.ds(base, BATCH)], col_v)
    _mask_cols(row_v, col_v, colm_v)
    pltpu.sync_copy(ones_v, acc.at[colm_v], add=True)
    return carry

  lax.fori_loop(0, nbatch, body, 0)
  plsc.subcore_barrier()
  pltpu.sync_copy(acc.at[my_rows], out.at[cid, my_rows])


def _agg_body(nbatch, epw, hp, ei, zeros128, out,
              row_v, col_v, colm_v, rows_v, sem, acc):
  cid = lax.axis_index("c")
  sid = lax.axis_index("s")
  wid = cid * NS + sid
  my_rows = pl.ds(sid * RPT, RPT)
  pltpu.sync_copy(zeros128, acc.at[my_rows])
  plsc.subcore_barrier()

  def body(b, carry):
    base = pl.multiple_of(wid * epw + b * BATCH, BATCH)
    pltpu.sync_copy(ei.at[0, pl.ds(base, BATCH)], row_v)
    pltpu.sync_copy(ei.at[1, pl.ds(base, BATCH)], col_v)
    _mask_cols(row_v, col_v, colm_v)
    pltpu.async_copy(hp.at[row_v], rows_v, sem).wait()
    pltpu.sync_copy(rows_v, acc.at[colm_v], add=True)
    return carry

  lax.fori_loop(0, nbatch, body, 0)
  plsc.subcore_barrier()
  pltpu.sync_copy(acc.at[my_rows], out.at[cid, my_rows])


def _deg_from_partials(degp):
  deg = jnp.sum(degp[0], axis=-1) + jnp.sum(degp[1], axis=-1) + 1.0
  return lax.rsqrt(deg)


def _mm_scale_body(x_ref, w_ref, degp_ref, out_ref):
  dis = _deg_from_partials(degp_ref[...])
  h = jnp.dot(x_ref[...], w_ref[...], preferred_element_type=jnp.float32)
  out_ref[...] = h * dis[:, None]


def _norm_body(aggp_ref, hp_ref, degp_ref, b_ref, g_ref, be_ref, out_ref):
  dis = _deg_from_partials(degp_ref[...])
  a = aggp_ref[0] + aggp_ref[1] + hp_ref[...]
  t = a * dis[:, None] + b_ref[...]
  r = jnp.maximum(t, 0.0)
  mu = jnp.mean(r, axis=-1, keepdims=True)
  c = r - mu
  var = jnp.mean(c * c, axis=-1, keepdims=True)
  out_ref[...] = c * lax.rsqrt(var + EPS) * g_ref[...] + be_ref[...]


def _make_sc_kernels(e_pad):
  epw = e_pad // NW
  nbatch = epw // BATCH
  deg_k = functools.partial(
      pl.kernel,
      out_type=jax.ShapeDtypeStruct((NC, N_ACC, D), jnp.float32),
      mesh=_mesh,
      scratch_types=[
          pltpu.VMEM((BATCH,), jnp.int32),
          pltpu.VMEM((BATCH,), jnp.int32),
          pltpu.VMEM((BATCH,), jnp.int32),
          pltpu.VMEM((BATCH, D), jnp.float32),
          pltpu.VMEM_SHARED((N_ACC, D), jnp.float32),
      ])(functools.partial(_deg_body, nbatch, epw))
  agg_k = functools.partial(
      pl.kernel,
      out_type=jax.ShapeDtypeStruct((NC, N_ACC, D), jnp.float32),
      mesh=_mesh,
      scratch_types=[
          pltpu.VMEM((BATCH,), jnp.int32),
          pltpu.VMEM((BATCH,), jnp.int32),
          pltpu.VMEM((BATCH,), jnp.int32),
          pltpu.VMEM((BATCH, D), jnp.float32),
          pltpu.SemaphoreType.DMA,
          pltpu.VMEM_SHARED((N_ACC, D), jnp.float32),
      ])(functools.partial(_agg_body, nbatch, epw))
  return deg_k, agg_k


def _tc_matmul_scale(x, w, degp):
  return pl.pallas_call(
      _mm_scale_body,
      grid=(N // BLK,),
      in_specs=[
          pl.BlockSpec((BLK, D), lambda i: (i, 0)),
          pl.BlockSpec((D, D), lambda i: (0, 0)),
          pl.BlockSpec((NC, BLK, D), lambda i: (0, i, 0)),
      ],
      out_specs=pl.BlockSpec((BLK, D), lambda i: (i, 0)),
      out_shape=jax.ShapeDtypeStruct((N, D), jnp.float32),
  )(x, w, degp)


def _tc_norm(aggp, hp, degp, b, g, be):
  return pl.pallas_call(
      _norm_body,
      grid=(N // BLK,),
      in_specs=[
          pl.BlockSpec((NC, BLK, D), lambda i: (0, i, 0)),
          pl.BlockSpec((BLK, D), lambda i: (i, 0)),
          pl.BlockSpec((NC, BLK, D), lambda i: (0, i, 0)),
          pl.BlockSpec((1, D), lambda i: (0, 0)),
          pl.BlockSpec((1, D), lambda i: (0, 0)),
          pl.BlockSpec((1, D), lambda i: (0, 0)),
      ],
      out_specs=pl.BlockSpec((BLK, D), lambda i: (i, 0)),
      out_shape=jax.ShapeDtypeStruct((N, D), jnp.float32),
  )(aggp, hp, degp, b, g, be)


def kernel(x, edge_index, W1, b1, g1, be1, W2, b2, g2, be2):
  e = edge_index.shape[1]
  e_pad = ((e + NW * BATCH - 1) // (NW * BATCH)) * (NW * BATCH)
  pad = e_pad - e
  ei = edge_index.astype(jnp.int32)
  if pad:
    fill = jnp.stack([jnp.zeros((pad,), jnp.int32),
                      jnp.full((pad,), DUMMY, jnp.int32)])
    ei = jnp.concatenate([ei, fill], axis=1)

  ones128 = jnp.full((BATCH, D), 1.0 / D, jnp.float32)
  zeros128 = jnp.zeros((RPT, D), jnp.float32)
  b1r, g1r, be1r = b1.reshape(1, D), g1.reshape(1, D), be1.reshape(1, D)
  b2r, g2r, be2r = b2.reshape(1, D), g2.reshape(1, D), be2.reshape(1, D)

  deg_k, agg_k = _make_sc_kernels(e_pad)

  degp = deg_k(ei, zeros128, ones128)

  h1 = _tc_matmul_scale(x, W1, degp)
  agg1 = agg_k(h1, ei, zeros128)
  z1 = _tc_norm(agg1, h1, degp, b1r, g1r, be1r)

  h2 = _tc_matmul_scale(z1, W2, degp)
  agg2 = agg_k(h2, ei, zeros128)
  z2 = _tc_norm(agg2, h2, degp, b2r, g2r, be2r)
  return z2
